# baseline (speedup 1.0000x reference)
import jax
import jax.numpy as jnp
from jax.experimental import pallas as pl

_BS = 1024


def _add_body(tok_ref, pos_ref, out_ref):
    out_ref[...] = tok_ref[...] + pos_ref[...][None, :, :]


def kernel(token_embeddings, pos_table):
    batch, seq, dim = token_embeddings.shape
    return pl.pallas_call(
        _add_body,
        grid=(seq // _BS, batch),
        in_specs=[
            pl.BlockSpec((1, _BS, dim), lambda s, b: (b, s, 0)),
            pl.BlockSpec((_BS, dim), lambda s, b: (s, 0)),
        ],
        out_specs=pl.BlockSpec((1, _BS, dim), lambda s, b: (b, s, 0)),
        out_shape=jax.ShapeDtypeStruct((batch, seq, dim), token_embeddings.dtype),
    )(token_embeddings, pos_table)


# speedup vs baseline: 1.0396x; 1.0396x over previous
import jax
import jax.numpy as jnp
from jax.experimental import pallas as pl

_BS = 2048


def _add_body(tok_ref, pos_ref, out_ref):
    out_ref[...] = tok_ref[...] + pos_ref[...][None, :, :]


def kernel(token_embeddings, pos_table):
    batch, seq, dim = token_embeddings.shape
    return pl.pallas_call(
        _add_body,
        grid=(seq // _BS, batch),
        in_specs=[
            pl.BlockSpec((1, _BS, dim), lambda s, b: (b, s, 0)),
            pl.BlockSpec((_BS, dim), lambda s, b: (s, 0)),
        ],
        out_specs=pl.BlockSpec((1, _BS, dim), lambda s, b: (b, s, 0)),
        out_shape=jax.ShapeDtypeStruct((batch, seq, dim), token_embeddings.dtype),
    )(token_embeddings, pos_table)
